# Initial kernel scaffold; baseline (speedup 1.0000x reference)
import jax
import jax.numpy as jnp
from jax.experimental import pallas as pl


def kernel(x, codebooks, cb_index):
    raise NotImplementedError("write your pallas kernel here")


# speedup vs baseline: 1.3030x; 1.3030x over previous
import jax
import jax.numpy as jnp
from jax import lax
from jax.experimental import pallas as pl
from jax.experimental.pallas import tpu as pltpu
from jax.experimental.pallas import tpu_sc as plsc

F = 26
B = 4096
MK = 1024
M = 4
PLEN = 16
N = B * F
NW = 32
NPW = N // NW
NE = NPW * M
OUTCH = NPW
NCHUNK = NE // OUTCH
DMA_I = 128
S1_DMAS = NE // DMA_I
S2_DMAS = OUTCH // DMA_I
PER = 208
PV = PER // 16


def _vperm(vec, idx):
    dnums = lax.GatherDimensionNumbers(
        offset_dims=(), collapsed_slice_dims=(0,), start_index_map=(0,))
    return lax.gather(vec, idx[:, None], dnums, (1,),
                      mode=lax.GatherScatterMode.PROMISE_IN_BOUNDS)


def _body(x_hbm, cbi_hbm, cb2_hbm, off1_hbm, off2_hbm, out_hbm,
          xv, idx1, cr, outv, o1v, o2v, sem1, sem2):
    nc = 2
    wid = lax.axis_index("s") * nc + lax.axis_index("c")
    base = wid * NPW

    pltpu.sync_copy(off1_hbm, o1v)
    pltpu.sync_copy(off2_hbm, o2v)
    pltpu.sync_copy(x_hbm.at[pl.ds(base, NPW)], xv)

    lane = lax.iota(jnp.int32, 16)
    rep4 = lax.shift_right_logical(lane, 2)

    def mk_idx1(t, _):
        xvec = xv[pl.ds(t * 16, 16)]
        for j in range(4):
            x4 = _vperm(xvec, rep4 + 4 * j)
            q = 4 * t + j
            p = 16 * lax.rem(q, PV)
            idx1[pl.ds(q * 16, 16)] = (x4 << 2) + o1v[pl.ds(p, 16)]
        return _
    lax.fori_loop(0, NPW // 16, mk_idx1, None)

    def s1_start(j, _):
        pltpu.make_async_copy(
            cbi_hbm.at[idx1.at[pl.ds(j * DMA_I, DMA_I)]],
            cr.at[pl.ds(j * DMA_I, DMA_I)], sem1).start()
        return _
    def s1_wait(j, _):
        pltpu.make_async_copy(
            cbi_hbm.at[idx1.at[pl.ds(j * DMA_I, DMA_I)]],
            cr.at[pl.ds(j * DMA_I, DMA_I)], sem1).wait()
        return _
    lax.fori_loop(0, S1_DMAS, s1_start, None)
    lax.fori_loop(0, S1_DMAS, s1_wait, None)

    def mk_idx2(q, _):
        p = 16 * lax.rem(q, PV)
        cr[pl.ds(q * 16, 16)] = (cr[pl.ds(q * 16, 16)] << 2) + o2v[pl.ds(p, 16)]
        return _
    lax.fori_loop(0, NE // 16, mk_idx2, None)

    for k in range(NCHUNK):
        def s2_start(j, _):
            pltpu.make_async_copy(
                cb2_hbm.at[cr.at[pl.ds(k * OUTCH + j * DMA_I, DMA_I)]],
                outv.at[pl.ds(j * DMA_I, DMA_I)], sem2).start()
            return _
        def s2_wait(j, _):
            pltpu.make_async_copy(
                cb2_hbm.at[cr.at[pl.ds(k * OUTCH + j * DMA_I, DMA_I)]],
                outv.at[pl.ds(j * DMA_I, DMA_I)], sem2).wait()
            return _
        lax.fori_loop(0, S2_DMAS, s2_start, None)
        lax.fori_loop(0, S2_DMAS, s2_wait, None)
        pltpu.sync_copy(outv, out_hbm.at[pl.ds(wid * NE + k * OUTCH, OUTCH)])


@jax.jit
def kernel(x, codebooks, cb_index):
    xf = x.reshape(N)
    cbi = cb_index.reshape(cb_index.shape[0] * M)
    cb2 = codebooks.reshape(F * MK * M, PLEN)
    ar = jnp.arange(PER, dtype=jnp.int32)
    off1 = ((ar // M) % F) * (4 * 40000) + (ar % M)
    off2 = ((ar // M) % F) * (MK * M) + (ar % M)

    run = pl.kernel(
        _body,
        out_type=jax.ShapeDtypeStruct((N * M, PLEN), jnp.float32),
        mesh=plsc.VectorSubcoreMesh(core_axis_name="c", subcore_axis_name="s"),
        scratch_types=[
            pltpu.VMEM((NPW,), jnp.int32),
            pltpu.VMEM((NE,), jnp.int32),
            pltpu.VMEM((NE,), jnp.int32),
            pltpu.VMEM((OUTCH, PLEN), jnp.float32),
            pltpu.VMEM((PER,), jnp.int32),
            pltpu.VMEM((PER,), jnp.int32),
            pltpu.SemaphoreType.DMA,
            pltpu.SemaphoreType.DMA,
        ],
        compiler_params=pltpu.CompilerParams(use_tc_tiling_on_sc=False),
    )
    out = run(xf, cbi, cb2, off1, off2)
    return out.reshape(B, F, M * PLEN)
